# Initial kernel scaffold; baseline (speedup 1.0000x reference)
import jax
import jax.numpy as jnp
from jax.experimental import pallas as pl


def kernel(img, hu_lis, norm_lis):
    raise NotImplementedError("write your pallas kernel here")


# speedup vs baseline: 1.9735x; 1.9735x over previous
import functools

import jax
import jax.numpy as jnp
from jax.experimental import pallas as pl
from jax.experimental.pallas import tpu as pltpu

_BASE_HU = -2.0
_BASE_NORM = 0.0

_SUB = 512


def _tc_body(hu_ref, norm_ref, x_ref, out_ref):
    x = x_ref[0, 0]
    for j in range(4):
        h_low = jnp.abs(hu_ref[j, 0])
        n_low = jnp.abs(norm_ref[j, 0])
        y = jnp.zeros_like(x)
        for i in range(1, 8):
            h_high = h_low + jnp.abs(hu_ref[j, i])
            n_high = n_low + jnp.abs(norm_ref[j, i])
            k = (n_high - n_low) / (h_high - h_low)
            a = k
            c = n_low - k * h_low
            y = jnp.where(x >= _BASE_HU + h_low, a * x + c, y)
            h_low, n_low = h_high, n_high
        y = jnp.where(x >= _BASE_HU + h_low, n_low + _BASE_NORM, y)
        out_ref[0, j, 0] = y


def kernel(img, hu_lis, norm_lis):
    B, C, D, H, W = img.shape
    total = D * H * W
    blk = _SUB * 256
    nblk = total // blk
    x = img.reshape(B, nblk, _SUB, 256)

    out = pl.pallas_call(
        _tc_body,
        grid=(B, nblk),
        in_specs=[
            pl.BlockSpec(memory_space=pltpu.SMEM),
            pl.BlockSpec(memory_space=pltpu.SMEM),
            pl.BlockSpec((1, 1, _SUB, 256), lambda b, i: (b, i, 0, 0)),
        ],
        out_specs=pl.BlockSpec((1, 4, 1, _SUB, 256), lambda b, i: (b, 0, i, 0, 0)),
        out_shape=jax.ShapeDtypeStruct((B, 4, nblk, _SUB, 256), jnp.float32),
    )(hu_lis, norm_lis, x)
    return out.reshape(B, 4, D, H, W)
